# baseline (speedup 1.0000x reference)
import functools

import jax
import jax.numpy as jnp
from jax.experimental import pallas as pl
from jax.experimental.pallas import tpu as pltpu

D_MODEL = 256
D_INNER = 512
D_STATE = 16
D_CONV = 4
DT_RANK = 16

T_CHUNK = 128
L_FULL = 4096
NC = L_FULL // T_CHUNK
T_CONV = 512
NCONV = L_FULL // T_CONV


def _silu(v):
    return v * (0.5 * jnp.tanh(0.5 * v) + 0.5)


def _convt_kernel(x_ref, w_ref, b_ref, o_ref):
    xm = x_ref[0].reshape(32 * 32, 256)
    res = jnp.dot(xm, w_ref[0], preferred_element_type=jnp.float32)
    res = res + b_ref[...]
    o_ref[0, 0] = res.reshape(32, 32, 256)


def _convt(x_t, w_stack, bias2):
    B = x_t.shape[0]
    return pl.pallas_call(
        _convt_kernel,
        grid=(B, 2),
        in_specs=[
            pl.BlockSpec((1, 32, 32, 256), lambda b, xp: (b, 0, 0, 0)),
            pl.BlockSpec((1, 256, 256), lambda b, xp: (xp, 0, 0)),
            pl.BlockSpec((1, 256), lambda b, xp: (0, 0)),
        ],
        out_specs=pl.BlockSpec((1, 1, 32, 32, 256), lambda b, xp: (b, xp, 0, 0, 0)),
        out_shape=jax.ShapeDtypeStruct((B, 2, 32, 32, 256), jnp.float32),
        compiler_params=pltpu.CompilerParams(
            dimension_semantics=("parallel", "arbitrary")),
        name="convt2x2",
    )(x_t, w_stack, bias2)


def _mamba_kernel(seq_ref, win_ref, wconv_ref, bconv_ref, wx_ref, wdt_ref,
                  bdt_ref, alogt_ref, q_ref,
                  d_ref, wout_ref, out_ref, xbuf, h_carry, da_s, db_s, hs_s):
    nc = pl.program_id(1)
    T = T_CHUNK

    seq = seq_ref[0]
    xz = jnp.dot(seq, win_ref[...], preferred_element_type=jnp.float32)
    xi = xz[:, :D_INNER]
    z = xz[:, D_INNER:]

    @pl.when(nc == 0)
    def _():
        xbuf[0:8, :] = jnp.zeros((8, D_INNER), jnp.float32)

    @pl.when(nc > 0)
    def _():
        xbuf[0:8, :] = xbuf[T:T + 8, :]

    xbuf[8:8 + T, :] = xi
    u_pre = bconv_ref[...]
    for k in range(D_CONV):
        u_pre = u_pre + wconv_ref[k:k + 1, :] * xbuf[8 - (D_CONV - 1) + k:
                                                     8 - (D_CONV - 1) + k + T, :]
    u = _silu(u_pre)

    dbc = jnp.dot(u, wx_ref[...], preferred_element_type=jnp.float32)
    dt_in = dbc[:, :DT_RANK]
    bm = dbc[:, DT_RANK:DT_RANK + D_STATE]
    cm = dbc[:, DT_RANK + D_STATE:]
    dt_pre = jnp.dot(dt_in, wdt_ref[...],
                     preferred_element_type=jnp.float32) + bdt_ref[...]
    dt = jnp.maximum(dt_pre, 0.0) + jnp.log1p(jnp.exp(-jnp.abs(dt_pre)))

    A = -jnp.exp(alogt_ref[...])
    da_s[...] = jnp.exp(dt[:, None, :] * A[None, :, :]).astype(jnp.bfloat16)
    db_s[...] = ((dt * u)[:, None, :] * bm[:, :, None]).astype(jnp.bfloat16)

    @pl.when(nc == 0)
    def _():
        h_carry[...] = jnp.zeros((D_STATE, D_INNER), jnp.bfloat16)

    def body(t, h):
        h = da_s[t] * h + db_s[t]
        hs_s[t] = h
        return h

    h_fin = jax.lax.fori_loop(0, T, body, h_carry[...])
    h_carry[...] = h_fin

    cm3 = jnp.broadcast_to(cm[:, :, None], (T, D_STATE, D_INNER))
    prod = (hs_s[...] * cm3.astype(jnp.bfloat16)).reshape(T * D_STATE, D_INNER)
    y_scan = jnp.dot(q_ref[...], prod, preferred_element_type=jnp.float32)
    y = (y_scan + u * d_ref[...]) * _silu(z)
    mam = jnp.dot(y, wout_ref[...], preferred_element_type=jnp.float32)
    out_ref[0] = seq + mam


def _mamba(seq, win_t, wconv_t, bconv, wx_t, wdt_t, bdt, alog_t, q_t,
           d_row, wout_t):
    B = seq.shape[0]
    return pl.pallas_call(
        _mamba_kernel,
        grid=(B, NC),
        in_specs=[
            pl.BlockSpec((1, T_CHUNK, 256), lambda b, i: (b, i, 0)),
            pl.BlockSpec((256, 1024), lambda b, i: (0, 0)),
            pl.BlockSpec((D_CONV, D_INNER), lambda b, i: (0, 0)),
            pl.BlockSpec((1, D_INNER), lambda b, i: (0, 0)),
            pl.BlockSpec((D_INNER, 48), lambda b, i: (0, 0)),
            pl.BlockSpec((DT_RANK, D_INNER), lambda b, i: (0, 0)),
            pl.BlockSpec((1, D_INNER), lambda b, i: (0, 0)),
            pl.BlockSpec((D_STATE, D_INNER), lambda b, i: (0, 0)),
            pl.BlockSpec((T_CHUNK, T_CHUNK * D_STATE), lambda b, i: (0, 0)),
            pl.BlockSpec((1, D_INNER), lambda b, i: (0, 0)),
            pl.BlockSpec((D_INNER, 256), lambda b, i: (0, 0)),
        ],
        out_specs=pl.BlockSpec((1, T_CHUNK, 256), lambda b, i: (b, i, 0)),
        out_shape=jax.ShapeDtypeStruct((B, L_FULL, 256), jnp.float32),
        scratch_shapes=[
            pltpu.VMEM((T_CHUNK + 8, D_INNER), jnp.float32),
            pltpu.VMEM((D_STATE, D_INNER), jnp.bfloat16),
            pltpu.VMEM((T_CHUNK, D_STATE, D_INNER), jnp.bfloat16),
            pltpu.VMEM((T_CHUNK, D_STATE, D_INNER), jnp.bfloat16),
            pltpu.VMEM((T_CHUNK, D_STATE, D_INNER), jnp.bfloat16),
        ],
        compiler_params=pltpu.CompilerParams(
            dimension_semantics=("parallel", "arbitrary"),
            vmem_limit_bytes=100 * 1024 * 1024),
        name="mamba_block",
    )(seq, win_t, wconv_t, bconv, wx_t, wdt_t, bdt, alog_t, q_t, d_row,
      wout_t)


def _conv3_kernel(x3_ref, w3_ref, b_ref, y2_ref, st_ref):
    R = 512
    bias = jnp.broadcast_to(b_ref[...], (R, 128))
    ssum = jnp.zeros((1, 128), jnp.float32)
    ssq = jnp.zeros((1, 128), jnp.float32)
    zpad = jnp.zeros((64, 128), jnp.float32)
    nr = L_FULL // R
    for r in range(nr):
        lo = r * R
        acc = bias + jnp.dot(x3_ref[0, lo:lo + R, :], w3_ref[1],
                             preferred_element_type=jnp.float32)
        if r == 0:
            top = jnp.dot(x3_ref[0, 0:R - 64, :], w3_ref[0],
                          preferred_element_type=jnp.float32)
            acc = acc + jnp.concatenate([zpad, top], axis=0)
        else:
            acc = acc + jnp.dot(x3_ref[0, lo - 64:lo + R - 64, :], w3_ref[0],
                                preferred_element_type=jnp.float32)
        if r == nr - 1:
            bot = jnp.dot(x3_ref[0, lo + 64:L_FULL, :], w3_ref[2],
                          preferred_element_type=jnp.float32)
            acc = acc + jnp.concatenate([bot, zpad], axis=0)
        else:
            acc = acc + jnp.dot(x3_ref[0, lo + 64:lo + R + 64, :], w3_ref[2],
                                preferred_element_type=jnp.float32)
        y2_ref[0, lo:lo + R, :] = acc
        ssum = ssum + jnp.sum(acc, axis=0, keepdims=True)
        ssq = ssq + jnp.sum(acc * acc, axis=0, keepdims=True)
    st_ref[0, 0:1, :] = ssum
    st_ref[0, 1:2, :] = ssq


def _conv3(x3, w3, bias):
    B = x3.shape[0]
    return pl.pallas_call(
        _conv3_kernel,
        grid=(B,),
        in_specs=[
            pl.BlockSpec((1, L_FULL, 768), lambda b: (b, 0, 0)),
            pl.BlockSpec((3, 768, 128), lambda b: (0, 0, 0)),
            pl.BlockSpec((1, 128), lambda b: (0, 0)),
        ],
        out_specs=[
            pl.BlockSpec((1, L_FULL, 128), lambda b: (b, 0, 0)),
            pl.BlockSpec((1, 2, 128), lambda b: (b, 0, 0)),
        ],
        out_shape=[
            jax.ShapeDtypeStruct((B, L_FULL, 128), jnp.float32),
            jax.ShapeDtypeStruct((B, 2, 128), jnp.float32),
        ],
        compiler_params=pltpu.CompilerParams(
            dimension_semantics=("parallel",),
            vmem_limit_bytes=100 * 1024 * 1024),
        name="conv3x3",
    )(x3, w3, bias)


def _bn_kernel(y2_ref, st_ref, g_ref, b_ref, o_ref):
    n = 2.0 * L_FULL
    tot = st_ref[0] + st_ref[1]
    mu = tot[0:1, :] / n
    var = tot[1:2, :] / n - mu * mu
    scale = jax.lax.rsqrt(var + 1e-5) * g_ref[...]
    shift = b_ref[...] - mu * scale
    yn = y2_ref[0] * scale + shift
    o_ref[0] = (yn * 0.5 * (1.0 + jax.lax.erf(yn * 0.7071067811865476))).T


def _bn_gelu(y2, stats, gamma, beta):
    B = y2.shape[0]
    return pl.pallas_call(
        _bn_kernel,
        grid=(B, NCONV),
        in_specs=[
            pl.BlockSpec((1, T_CONV, 128), lambda b, i: (b, i, 0)),
            pl.BlockSpec((2, 2, 128), lambda b, i: (0, 0, 0)),
            pl.BlockSpec((1, 128), lambda b, i: (0, 0)),
            pl.BlockSpec((1, 128), lambda b, i: (0, 0)),
        ],
        out_specs=pl.BlockSpec((1, 128, T_CONV), lambda b, i: (b, 0, i)),
        out_shape=jax.ShapeDtypeStruct((B, 128, L_FULL), jnp.float32),
        compiler_params=pltpu.CompilerParams(
            dimension_semantics=("parallel", "arbitrary")),
        name="bn_gelu",
    )(y2, stats, gamma, beta)


@jax.jit
def kernel(x, skip, up_w, up_b, in_proj_w, conv1d_w, conv1d_b, x_proj_w,
           dt_proj_w, dt_proj_b, A_log, D, out_proj_w, conv_w, conv_b,
           bn_gamma, bn_beta):
    B = x.shape[0]

    x_t = x.transpose(0, 2, 3, 1)
    w_stack = jnp.stack([
        up_w[:, :, 0, :].transpose(0, 2, 1).reshape(256, 256),
        up_w[:, :, 1, :].transpose(0, 2, 1).reshape(256, 256),
    ])
    bias2 = jnp.tile(up_b, 2).reshape(1, 256)
    up5 = _convt(x_t, w_stack, bias2)
    up_nhwc = (up5.reshape(B, 2, 32, 32, 2, 128)
               .transpose(0, 2, 1, 3, 4, 5)
               .reshape(B, 64, 64, 128))
    skip_t = skip.transpose(0, 2, 3, 1)
    xc = jnp.concatenate([up_nhwc, skip_t], axis=-1)
    seq = xc.reshape(B, L_FULL, 256)

    xr = _mamba(
        seq,
        in_proj_w.T,
        conv1d_w.T,
        conv1d_b.reshape(1, D_INNER),
        x_proj_w.T,
        dt_proj_w.T,
        dt_proj_b.reshape(1, D_INNER),
        A_log.T,
        jnp.repeat(jnp.eye(T_CHUNK, dtype=jnp.bfloat16), D_STATE, axis=1),
        D.reshape(1, D_INNER),
        out_proj_w.T,
    )

    xr_im = xr.reshape(B, 64, 64, 256)
    xr_wp = jnp.pad(xr_im, ((0, 0), (0, 0), (1, 1), (0, 0)))
    x3 = jnp.concatenate(
        [xr_wp[:, :, dj:dj + 64, :] for dj in range(3)], axis=-1)
    x3 = x3.reshape(B, L_FULL, 768)
    w3 = conv_w.transpose(2, 3, 1, 0).reshape(3, 768, 128)
    y2, stats = _conv3(x3, w3, conv_b.reshape(1, 128))

    out = _bn_gelu(y2, stats, bn_gamma.reshape(1, 128),
                   bn_beta.reshape(1, 128))
    return out.reshape(B, 128, 64, 64)


# speedup vs baseline: 1.1309x; 1.1309x over previous
import functools

import jax
import jax.numpy as jnp
from jax.experimental import pallas as pl
from jax.experimental.pallas import tpu as pltpu

D_MODEL = 256
D_INNER = 512
D_STATE = 16
D_CONV = 4
DT_RANK = 16

T_CHUNK = 128
L_FULL = 4096
NC = L_FULL // T_CHUNK
T_CONV = 512
NCONV = L_FULL // T_CONV


def _silu(v):
    return v * (0.5 * jnp.tanh(0.5 * v) + 0.5)


def _convt_kernel(x_ref, w_ref, b_ref, o_ref):
    xm = x_ref[0].reshape(32 * 32, 256)
    res = jnp.dot(xm, w_ref[0], preferred_element_type=jnp.float32)
    res = res + b_ref[...]
    o_ref[0, 0] = res.reshape(32, 32, 256)


def _convt(x_t, w_stack, bias2):
    B = x_t.shape[0]
    return pl.pallas_call(
        _convt_kernel,
        grid=(B, 2),
        in_specs=[
            pl.BlockSpec((1, 32, 32, 256), lambda b, xp: (b, 0, 0, 0)),
            pl.BlockSpec((1, 256, 256), lambda b, xp: (xp, 0, 0)),
            pl.BlockSpec((1, 256), lambda b, xp: (0, 0)),
        ],
        out_specs=pl.BlockSpec((1, 1, 32, 32, 256), lambda b, xp: (b, xp, 0, 0, 0)),
        out_shape=jax.ShapeDtypeStruct((B, 2, 32, 32, 256), jnp.float32),
        compiler_params=pltpu.CompilerParams(
            dimension_semantics=("parallel", "arbitrary")),
        name="convt2x2",
    )(x_t, w_stack, bias2)


def _mamba_kernel(up_ref, sk_ref, win_ref, wconv_ref, bconv_ref, wx_ref,
                  wdt_ref, bdt_ref, alogt_ref, q_ref,
                  d_ref, wout_ref, out_ref, xbuf, h_carry, da_s, db_s, hs_s):
    nc = pl.program_id(1)
    T = T_CHUNK

    seq = jnp.concatenate([up_ref[0], sk_ref[0]], axis=-1)
    xz = jnp.dot(seq, win_ref[...], preferred_element_type=jnp.float32)
    xi = xz[:, :D_INNER]
    z = xz[:, D_INNER:]

    @pl.when(nc == 0)
    def _():
        xbuf[0:8, :] = jnp.zeros((8, D_INNER), jnp.float32)

    @pl.when(nc > 0)
    def _():
        xbuf[0:8, :] = xbuf[T:T + 8, :]

    xbuf[8:8 + T, :] = xi
    u_pre = bconv_ref[...]
    for k in range(D_CONV):
        u_pre = u_pre + wconv_ref[k:k + 1, :] * xbuf[8 - (D_CONV - 1) + k:
                                                     8 - (D_CONV - 1) + k + T, :]
    u = _silu(u_pre)

    dbc = jnp.dot(u, wx_ref[...], preferred_element_type=jnp.float32)
    dt_in = dbc[:, :DT_RANK]
    bm = dbc[:, DT_RANK:DT_RANK + D_STATE]
    cm = dbc[:, DT_RANK + D_STATE:]
    dt_pre = jnp.dot(dt_in, wdt_ref[...],
                     preferred_element_type=jnp.float32) + bdt_ref[...]
    dt = jnp.maximum(dt_pre, 0.0) + jnp.log1p(jnp.exp(-jnp.abs(dt_pre)))

    A = -jnp.exp(alogt_ref[...])
    da_s[...] = jnp.exp(dt[:, None, :] * A[None, :, :]).astype(jnp.bfloat16)
    db_s[...] = ((dt * u)[:, None, :] * bm[:, :, None]).astype(jnp.bfloat16)

    @pl.when(nc == 0)
    def _():
        h_carry[...] = jnp.zeros((D_STATE, D_INNER), jnp.bfloat16)

    def body(t, h):
        h = da_s[t] * h + db_s[t]
        hs_s[t] = h
        return h

    h_fin = jax.lax.fori_loop(0, T, body, h_carry[...], unroll=4)
    h_carry[...] = h_fin

    cm3 = jnp.broadcast_to(cm[:, :, None], (T, D_STATE, D_INNER))
    prod = (hs_s[...] * cm3.astype(jnp.bfloat16)).reshape(T * D_STATE, D_INNER)
    y_scan = jnp.dot(q_ref[...], prod, preferred_element_type=jnp.float32)
    y = (y_scan + u * d_ref[...]) * _silu(z)
    mam = jnp.dot(y, wout_ref[...], preferred_element_type=jnp.float32)
    out_ref[0] = seq + mam


def _mamba(up_l, sk_l, win_t, wconv_t, bconv, wx_t, wdt_t, bdt, alog_t, q_t,
           d_row, wout_t):
    B = up_l.shape[0]
    return pl.pallas_call(
        _mamba_kernel,
        grid=(B, NC),
        in_specs=[
            pl.BlockSpec((1, T_CHUNK, 128), lambda b, i: (b, i, 0)),
            pl.BlockSpec((1, T_CHUNK, 128), lambda b, i: (b, i, 0)),
            pl.BlockSpec((256, 1024), lambda b, i: (0, 0)),
            pl.BlockSpec((D_CONV, D_INNER), lambda b, i: (0, 0)),
            pl.BlockSpec((1, D_INNER), lambda b, i: (0, 0)),
            pl.BlockSpec((D_INNER, 48), lambda b, i: (0, 0)),
            pl.BlockSpec((DT_RANK, D_INNER), lambda b, i: (0, 0)),
            pl.BlockSpec((1, D_INNER), lambda b, i: (0, 0)),
            pl.BlockSpec((D_STATE, D_INNER), lambda b, i: (0, 0)),
            pl.BlockSpec((T_CHUNK, T_CHUNK * D_STATE), lambda b, i: (0, 0)),
            pl.BlockSpec((1, D_INNER), lambda b, i: (0, 0)),
            pl.BlockSpec((D_INNER, 256), lambda b, i: (0, 0)),
        ],
        out_specs=pl.BlockSpec((1, T_CHUNK, 256), lambda b, i: (b, i, 0)),
        out_shape=jax.ShapeDtypeStruct((B, L_FULL, 256), jnp.float32),
        scratch_shapes=[
            pltpu.VMEM((T_CHUNK + 8, D_INNER), jnp.float32),
            pltpu.VMEM((D_STATE, D_INNER), jnp.bfloat16),
            pltpu.VMEM((T_CHUNK, D_STATE, D_INNER), jnp.bfloat16),
            pltpu.VMEM((T_CHUNK, D_STATE, D_INNER), jnp.bfloat16),
            pltpu.VMEM((T_CHUNK, D_STATE, D_INNER), jnp.bfloat16),
        ],
        compiler_params=pltpu.CompilerParams(
            dimension_semantics=("parallel", "arbitrary"),
            vmem_limit_bytes=100 * 1024 * 1024),
        name="mamba_block",
    )(up_l, sk_l, win_t, wconv_t, bconv, wx_t, wdt_t, bdt, alog_t, q_t,
      d_row, wout_t)


def _conv3_kernel(x3_ref, w3_ref, b_ref, y2_ref, st_ref):
    R = 512
    bias = jnp.broadcast_to(b_ref[...], (R, 128))
    ssum = jnp.zeros((1, 128), jnp.float32)
    ssq = jnp.zeros((1, 128), jnp.float32)
    zpad = jnp.zeros((64, 128), jnp.float32)
    nr = L_FULL // R
    for r in range(nr):
        lo = r * R
        acc = bias + jnp.dot(x3_ref[0, lo:lo + R, :], w3_ref[1],
                             preferred_element_type=jnp.float32)
        if r == 0:
            top = jnp.dot(x3_ref[0, 0:R - 64, :], w3_ref[0],
                          preferred_element_type=jnp.float32)
            acc = acc + jnp.concatenate([zpad, top], axis=0)
        else:
            acc = acc + jnp.dot(x3_ref[0, lo - 64:lo + R - 64, :], w3_ref[0],
                                preferred_element_type=jnp.float32)
        if r == nr - 1:
            bot = jnp.dot(x3_ref[0, lo + 64:L_FULL, :], w3_ref[2],
                          preferred_element_type=jnp.float32)
            acc = acc + jnp.concatenate([bot, zpad], axis=0)
        else:
            acc = acc + jnp.dot(x3_ref[0, lo + 64:lo + R + 64, :], w3_ref[2],
                                preferred_element_type=jnp.float32)
        y2_ref[0, lo:lo + R, :] = acc
        ssum = ssum + jnp.sum(acc, axis=0, keepdims=True)
        ssq = ssq + jnp.sum(acc * acc, axis=0, keepdims=True)
    st_ref[0, 0:1, :] = ssum
    st_ref[0, 1:2, :] = ssq


def _conv3(x3, w3, bias):
    B = x3.shape[0]
    return pl.pallas_call(
        _conv3_kernel,
        grid=(B,),
        in_specs=[
            pl.BlockSpec((1, L_FULL, 768), lambda b: (b, 0, 0)),
            pl.BlockSpec((3, 768, 128), lambda b: (0, 0, 0)),
            pl.BlockSpec((1, 128), lambda b: (0, 0)),
        ],
        out_specs=[
            pl.BlockSpec((1, L_FULL, 128), lambda b: (b, 0, 0)),
            pl.BlockSpec((1, 2, 128), lambda b: (b, 0, 0)),
        ],
        out_shape=[
            jax.ShapeDtypeStruct((B, L_FULL, 128), jnp.float32),
            jax.ShapeDtypeStruct((B, 2, 128), jnp.float32),
        ],
        compiler_params=pltpu.CompilerParams(
            dimension_semantics=("parallel",),
            vmem_limit_bytes=100 * 1024 * 1024),
        name="conv3x3",
    )(x3, w3, bias)


def _bn_kernel(y2_ref, st_ref, g_ref, b_ref, o_ref):
    n = 2.0 * L_FULL
    tot = st_ref[0] + st_ref[1]
    mu = tot[0:1, :] / n
    var = tot[1:2, :] / n - mu * mu
    scale = jax.lax.rsqrt(var + 1e-5) * g_ref[...]
    shift = b_ref[...] - mu * scale
    yn = y2_ref[0] * scale + shift
    o_ref[0] = yn * 0.5 * (1.0 + jax.lax.erf(yn * 0.7071067811865476))


def _bn_gelu(y2, stats, gamma, beta):
    B = y2.shape[0]
    return pl.pallas_call(
        _bn_kernel,
        grid=(B, NCONV),
        in_specs=[
            pl.BlockSpec((1, T_CONV, 128), lambda b, i: (b, i, 0)),
            pl.BlockSpec((2, 2, 128), lambda b, i: (0, 0, 0)),
            pl.BlockSpec((1, 128), lambda b, i: (0, 0)),
            pl.BlockSpec((1, 128), lambda b, i: (0, 0)),
        ],
        out_specs=pl.BlockSpec((1, T_CONV, 128), lambda b, i: (b, i, 0)),
        out_shape=jax.ShapeDtypeStruct((B, L_FULL, 128), jnp.float32),
        compiler_params=pltpu.CompilerParams(
            dimension_semantics=("parallel", "arbitrary")),
        name="bn_gelu",
    )(y2, stats, gamma, beta)


@jax.jit
def kernel(x, skip, up_w, up_b, in_proj_w, conv1d_w, conv1d_b, x_proj_w,
           dt_proj_w, dt_proj_b, A_log, D, out_proj_w, conv_w, conv_b,
           bn_gamma, bn_beta):
    B = x.shape[0]

    x_t = x.transpose(0, 2, 3, 1)
    w_stack = jnp.stack([
        up_w[:, :, 0, :].transpose(0, 2, 1).reshape(256, 256),
        up_w[:, :, 1, :].transpose(0, 2, 1).reshape(256, 256),
    ])
    bias2 = jnp.tile(up_b, 2).reshape(1, 256)
    up5 = _convt(x_t, w_stack, bias2)
    up_l = (up5.reshape(B, 2, 32, 32, 2, 128)
            .transpose(0, 2, 1, 3, 4, 5)
            .reshape(B, L_FULL, 128))
    sk_l = skip.transpose(0, 2, 3, 1).reshape(B, L_FULL, 128)

    xr = _mamba(
        up_l,
        sk_l,
        in_proj_w.T,
        conv1d_w.T,
        conv1d_b.reshape(1, D_INNER),
        x_proj_w.T,
        dt_proj_w.T,
        dt_proj_b.reshape(1, D_INNER),
        A_log.T,
        jnp.repeat(jnp.eye(T_CHUNK, dtype=jnp.bfloat16), D_STATE, axis=1),
        D.reshape(1, D_INNER),
        out_proj_w.T,
    )

    xr_im = xr.reshape(B, 64, 64, 256)
    xr_wp = jnp.pad(xr_im, ((0, 0), (0, 0), (1, 1), (0, 0)))
    x3 = jnp.concatenate(
        [xr_wp[:, :, dj:dj + 64, :] for dj in range(3)], axis=-1)
    x3 = x3.reshape(B, L_FULL, 768)
    w3 = conv_w.transpose(2, 3, 1, 0).reshape(3, 768, 128)
    y2, stats = _conv3(x3, w3, conv_b.reshape(1, 128))

    out = _bn_gelu(y2, stats, bn_gamma.reshape(1, 128),
                   bn_beta.reshape(1, 128))
    return out.reshape(B, 64, 64, 128).transpose(0, 3, 1, 2)


# speedup vs baseline: 1.2306x; 1.0881x over previous
import functools

import jax
import jax.numpy as jnp
from jax.experimental import pallas as pl
from jax.experimental.pallas import tpu as pltpu

D_MODEL = 256
D_INNER = 512
D_STATE = 16
D_CONV = 4
DT_RANK = 16

T_CHUNK = 128
L_FULL = 4096
NC = L_FULL // T_CHUNK
T_CONV = 512
NCONV = L_FULL // T_CONV


def _silu(v):
    return v * (0.5 * jnp.tanh(0.5 * v) + 0.5)


def _convt_kernel(x_ref, w_ref, b_ref, o_ref):
    xm = x_ref[0].reshape(32 * 32, 256)
    res = jnp.dot(xm, w_ref[0], preferred_element_type=jnp.float32)
    res = res + b_ref[...]
    o_ref[0, 0] = res.reshape(32, 32, 256)


def _convt(x_t, w_stack, bias2):
    B = x_t.shape[0]
    return pl.pallas_call(
        _convt_kernel,
        grid=(B, 2),
        in_specs=[
            pl.BlockSpec((1, 32, 32, 256), lambda b, xp: (b, 0, 0, 0)),
            pl.BlockSpec((1, 256, 256), lambda b, xp: (xp, 0, 0)),
            pl.BlockSpec((1, 256), lambda b, xp: (0, 0)),
        ],
        out_specs=pl.BlockSpec((1, 1, 32, 32, 256), lambda b, xp: (b, xp, 0, 0, 0)),
        out_shape=jax.ShapeDtypeStruct((B, 2, 32, 32, 256), jnp.float32),
        compiler_params=pltpu.CompilerParams(
            dimension_semantics=("parallel", "arbitrary")),
        name="convt2x2",
    )(x_t, w_stack, bias2)


def _mamba_kernel(up_ref, sk_ref, win_ref, wconv_ref, bconv_ref, wx_ref,
                  wdt_ref, bdt_ref, alogt_ref, q_ref,
                  d_ref, wout_ref, out_ref, xbuf, h_carry, da_s, db_s, hs_s):
    nc = pl.program_id(1)
    T = T_CHUNK

    seq = jnp.concatenate([up_ref[0], sk_ref[0]], axis=-1)
    xz = jnp.dot(seq, win_ref[...], preferred_element_type=jnp.float32)
    xi = xz[:, :D_INNER]
    z = xz[:, D_INNER:]

    @pl.when(nc == 0)
    def _():
        xbuf[0:8, :] = jnp.zeros((8, D_INNER), jnp.float32)

    @pl.when(nc > 0)
    def _():
        xbuf[0:8, :] = xbuf[T:T + 8, :]

    xbuf[8:8 + T, :] = xi
    u_pre = bconv_ref[...]
    for k in range(D_CONV):
        u_pre = u_pre + wconv_ref[k:k + 1, :] * xbuf[8 - (D_CONV - 1) + k:
                                                     8 - (D_CONV - 1) + k + T, :]
    u = _silu(u_pre)

    dbc = jnp.dot(u, wx_ref[...], preferred_element_type=jnp.float32)
    dt_in = dbc[:, :DT_RANK]
    bm = dbc[:, DT_RANK:DT_RANK + D_STATE]
    cm = dbc[:, DT_RANK + D_STATE:]
    dt_pre = jnp.dot(dt_in, wdt_ref[...],
                     preferred_element_type=jnp.float32) + bdt_ref[...]
    dt = jnp.maximum(dt_pre, 0.0) + jnp.log1p(jnp.exp(-jnp.abs(dt_pre)))

    A = -jnp.exp(alogt_ref[...])
    da_s[...] = jnp.exp(dt[:, None, :] * A[None, :, :]).astype(jnp.bfloat16)
    db_s[...] = ((dt * u)[:, None, :] * bm[:, :, None]).astype(jnp.bfloat16)

    @pl.when(nc == 0)
    def _():
        h_carry[...] = jnp.zeros((D_STATE, D_INNER), jnp.bfloat16)

    def body(t, h):
        h = da_s[t] * h + db_s[t]
        hs_s[t] = h
        return h

    h_fin = jax.lax.fori_loop(0, T, body, h_carry[...], unroll=4)
    h_carry[...] = h_fin

    cm3 = jnp.broadcast_to(cm[:, :, None], (T, D_STATE, D_INNER))
    prod = (hs_s[...] * cm3.astype(jnp.bfloat16)).reshape(T * D_STATE, D_INNER)
    y_scan = jnp.dot(q_ref[...], prod, preferred_element_type=jnp.float32)
    y = (y_scan + u * d_ref[...]) * _silu(z)
    mam = jnp.dot(y, wout_ref[...], preferred_element_type=jnp.float32)
    out_ref[0] = seq + mam


def _mamba(up_l, sk_l, win_t, wconv_t, bconv, wx_t, wdt_t, bdt, alog_t, q_t,
           d_row, wout_t):
    B = up_l.shape[0]
    return pl.pallas_call(
        _mamba_kernel,
        grid=(B, NC),
        in_specs=[
            pl.BlockSpec((1, T_CHUNK, 128), lambda b, i: (b, i, 0)),
            pl.BlockSpec((1, T_CHUNK, 128), lambda b, i: (b, i, 0)),
            pl.BlockSpec((256, 1024), lambda b, i: (0, 0)),
            pl.BlockSpec((D_CONV, D_INNER), lambda b, i: (0, 0)),
            pl.BlockSpec((1, D_INNER), lambda b, i: (0, 0)),
            pl.BlockSpec((D_INNER, 48), lambda b, i: (0, 0)),
            pl.BlockSpec((DT_RANK, D_INNER), lambda b, i: (0, 0)),
            pl.BlockSpec((1, D_INNER), lambda b, i: (0, 0)),
            pl.BlockSpec((D_STATE, D_INNER), lambda b, i: (0, 0)),
            pl.BlockSpec((T_CHUNK, T_CHUNK * D_STATE), lambda b, i: (0, 0)),
            pl.BlockSpec((1, D_INNER), lambda b, i: (0, 0)),
            pl.BlockSpec((D_INNER, 256), lambda b, i: (0, 0)),
        ],
        out_specs=pl.BlockSpec((1, T_CHUNK, 256), lambda b, i: (b, i, 0)),
        out_shape=jax.ShapeDtypeStruct((B, L_FULL, 256), jnp.float32),
        scratch_shapes=[
            pltpu.VMEM((T_CHUNK + 8, D_INNER), jnp.float32),
            pltpu.VMEM((D_STATE, D_INNER), jnp.bfloat16),
            pltpu.VMEM((T_CHUNK, D_STATE, D_INNER), jnp.bfloat16),
            pltpu.VMEM((T_CHUNK, D_STATE, D_INNER), jnp.bfloat16),
            pltpu.VMEM((T_CHUNK, D_STATE, D_INNER), jnp.bfloat16),
        ],
        compiler_params=pltpu.CompilerParams(
            dimension_semantics=("parallel", "arbitrary"),
            vmem_limit_bytes=100 * 1024 * 1024),
        name="mamba_block",
    )(up_l, sk_l, win_t, wconv_t, bconv, wx_t, wdt_t, bdt, alog_t, q_t,
      d_row, wout_t)


def _conv3_kernel(x_ref, w9_ref, b_ref, me_ref, y2_ref, st_ref):
    R = 512
    bias = jnp.broadcast_to(b_ref[...], (R, 128))
    ssum = jnp.zeros((1, 128), jnp.float32)
    ssq = jnp.zeros((1, 128), jnp.float32)
    nr = L_FULL // R
    for r in range(nr):
        lo = r * R
        acc = bias
        for di in range(3):
            for dj in range(3):
                off = lo + 64 * (di - 1) + (dj - 1)
                s0, s1 = off, off + R
                c0, c1 = max(0, -s0), R - max(0, s1 - L_FULL)
                lhs = x_ref[0, s0 + c0:s1 - (R - c1), :]
                if dj == 0:
                    lhs = lhs * me_ref[0, c0:c1, :]
                elif dj == 2:
                    lhs = lhs * me_ref[1, c0:c1, :]
                part = jnp.dot(lhs, w9_ref[3 * di + dj],
                               preferred_element_type=jnp.float32)
                if c0 > 0 or c1 < R:
                    pads = []
                    if c0 > 0:
                        pads.append(jnp.zeros((c0, 128), jnp.float32))
                    pads.append(part)
                    if c1 < R:
                        pads.append(jnp.zeros((R - c1, 128), jnp.float32))
                    part = jnp.concatenate(pads, axis=0)
                acc = acc + part
        y2_ref[0, lo:lo + R, :] = acc
        ssum = ssum + jnp.sum(acc, axis=0, keepdims=True)
        ssq = ssq + jnp.sum(acc * acc, axis=0, keepdims=True)
    st_ref[0, 0:1, :] = ssum
    st_ref[0, 1:2, :] = ssq


def _conv3(xr, w9, bias, medge):
    B = xr.shape[0]
    return pl.pallas_call(
        _conv3_kernel,
        grid=(B,),
        in_specs=[
            pl.BlockSpec((1, L_FULL, 256), lambda b: (b, 0, 0)),
            pl.BlockSpec((9, 256, 128), lambda b: (0, 0, 0)),
            pl.BlockSpec((1, 128), lambda b: (0, 0)),
            pl.BlockSpec((2, 512, 256), lambda b: (0, 0, 0)),
        ],
        out_specs=[
            pl.BlockSpec((1, L_FULL, 128), lambda b: (b, 0, 0)),
            pl.BlockSpec((1, 2, 128), lambda b: (b, 0, 0)),
        ],
        out_shape=[
            jax.ShapeDtypeStruct((B, L_FULL, 128), jnp.float32),
            jax.ShapeDtypeStruct((B, 2, 128), jnp.float32),
        ],
        compiler_params=pltpu.CompilerParams(
            dimension_semantics=("parallel",),
            vmem_limit_bytes=100 * 1024 * 1024),
        name="conv3x3",
    )(xr, w9, bias, medge)


def _bn_kernel(y2_ref, st_ref, g_ref, b_ref, o_ref):
    n = 2.0 * L_FULL
    tot = st_ref[0] + st_ref[1]
    mu = tot[0:1, :] / n
    var = tot[1:2, :] / n - mu * mu
    scale = jax.lax.rsqrt(var + 1e-5) * g_ref[...]
    shift = b_ref[...] - mu * scale
    yn = y2_ref[0] * scale + shift
    o_ref[0] = yn * 0.5 * (1.0 + jax.lax.erf(yn * 0.7071067811865476))


def _bn_gelu(y2, stats, gamma, beta):
    B = y2.shape[0]
    return pl.pallas_call(
        _bn_kernel,
        grid=(B, NCONV),
        in_specs=[
            pl.BlockSpec((1, T_CONV, 128), lambda b, i: (b, i, 0)),
            pl.BlockSpec((2, 2, 128), lambda b, i: (0, 0, 0)),
            pl.BlockSpec((1, 128), lambda b, i: (0, 0)),
            pl.BlockSpec((1, 128), lambda b, i: (0, 0)),
        ],
        out_specs=pl.BlockSpec((1, T_CONV, 128), lambda b, i: (b, i, 0)),
        out_shape=jax.ShapeDtypeStruct((B, L_FULL, 128), jnp.float32),
        compiler_params=pltpu.CompilerParams(
            dimension_semantics=("parallel", "arbitrary")),
        name="bn_gelu",
    )(y2, stats, gamma, beta)


@jax.jit
def kernel(x, skip, up_w, up_b, in_proj_w, conv1d_w, conv1d_b, x_proj_w,
           dt_proj_w, dt_proj_b, A_log, D, out_proj_w, conv_w, conv_b,
           bn_gamma, bn_beta):
    B = x.shape[0]

    x_t = x.transpose(0, 2, 3, 1)
    w_stack = jnp.stack([
        up_w[:, :, 0, :].transpose(0, 2, 1).reshape(256, 256),
        up_w[:, :, 1, :].transpose(0, 2, 1).reshape(256, 256),
    ])
    bias2 = jnp.tile(up_b, 2).reshape(1, 256)
    up5 = _convt(x_t, w_stack, bias2)
    up_l = (up5.reshape(B, 2, 32, 32, 2, 128)
            .transpose(0, 2, 1, 3, 4, 5)
            .reshape(B, L_FULL, 128))
    sk_l = skip.transpose(0, 2, 3, 1).reshape(B, L_FULL, 128)

    xr = _mamba(
        up_l,
        sk_l,
        in_proj_w.T,
        conv1d_w.T,
        conv1d_b.reshape(1, D_INNER),
        x_proj_w.T,
        dt_proj_w.T,
        dt_proj_b.reshape(1, D_INNER),
        A_log.T,
        jnp.repeat(jnp.eye(T_CHUNK, dtype=jnp.bfloat16), D_STATE, axis=1),
        D.reshape(1, D_INNER),
        out_proj_w.T,
    )

    rows = jnp.arange(512, dtype=jnp.int32) % 64
    medge = jnp.stack([
        jnp.broadcast_to((rows != 0).astype(jnp.float32)[:, None], (512, 256)),
        jnp.broadcast_to((rows != 63).astype(jnp.float32)[:, None], (512, 256)),
    ])
    w9 = conv_w.transpose(2, 3, 1, 0).reshape(9, 256, 128)
    y2, stats = _conv3(xr, w9, conv_b.reshape(1, 128), medge)

    out = _bn_gelu(y2, stats, bn_gamma.reshape(1, 128),
                   bn_beta.reshape(1, 128))
    return out.reshape(B, 64, 64, 128).transpose(0, 3, 1, 2)


# speedup vs baseline: 1.2505x; 1.0162x over previous
import functools

import jax
import jax.numpy as jnp
from jax.experimental import pallas as pl
from jax.experimental.pallas import tpu as pltpu

D_MODEL = 256
D_INNER = 512
D_STATE = 16
D_CONV = 4
DT_RANK = 16

T_CHUNK = 128
L_FULL = 4096
NC = L_FULL // T_CHUNK
T_CONV = 512
NCONV = L_FULL // T_CONV


def _silu(v):
    return v * (0.5 * jnp.tanh(0.5 * v) + 0.5)


def _convt_kernel(x_ref, w_ref, b_ref, o_ref):
    xm = x_ref[0].reshape(32 * 32, 256)
    res = jnp.dot(xm, w_ref[0], preferred_element_type=jnp.float32)
    res = res + b_ref[...]
    o_ref[0, 0] = res.reshape(32, 32, 256)


def _convt(x_t, w_stack, bias2):
    B = x_t.shape[0]
    return pl.pallas_call(
        _convt_kernel,
        grid=(B, 2),
        in_specs=[
            pl.BlockSpec((1, 32, 32, 256), lambda b, xp: (b, 0, 0, 0)),
            pl.BlockSpec((1, 256, 256), lambda b, xp: (xp, 0, 0)),
            pl.BlockSpec((1, 256), lambda b, xp: (0, 0)),
        ],
        out_specs=pl.BlockSpec((1, 1, 32, 32, 256), lambda b, xp: (b, xp, 0, 0, 0)),
        out_shape=jax.ShapeDtypeStruct((B, 2, 32, 32, 256), jnp.float32),
        compiler_params=pltpu.CompilerParams(
            dimension_semantics=("parallel", "arbitrary")),
        name="convt2x2",
    )(x_t, w_stack, bias2)


def _mamba_kernel(up_ref, sk_ref, win_ref, wconv_ref, bconv_ref, wx_ref,
                  wdt_ref, bdt_ref, alogt_ref, q_ref,
                  d_ref, wout_ref, out_ref, xbuf, h_carry, da_s, db_s, hs_s):
    nc = pl.program_id(1)
    T = T_CHUNK

    seq = jnp.concatenate([up_ref[0], sk_ref[0]], axis=-1)
    xz = jnp.dot(seq, win_ref[...], preferred_element_type=jnp.float32)
    xi = xz[:, :D_INNER]
    z = xz[:, D_INNER:]

    @pl.when(nc == 0)
    def _():
        xbuf[0:8, :] = jnp.zeros((8, D_INNER), jnp.float32)

    @pl.when(nc > 0)
    def _():
        xbuf[0:8, :] = xbuf[T:T + 8, :]

    xbuf[8:8 + T, :] = xi
    u_pre = bconv_ref[...]
    for k in range(D_CONV):
        u_pre = u_pre + wconv_ref[k:k + 1, :] * xbuf[8 - (D_CONV - 1) + k:
                                                     8 - (D_CONV - 1) + k + T, :]
    u = _silu(u_pre)

    dbc = jnp.dot(u, wx_ref[...], preferred_element_type=jnp.float32)
    dt_in = dbc[:, :DT_RANK]
    bm = dbc[:, DT_RANK:DT_RANK + D_STATE]
    cm = dbc[:, DT_RANK + D_STATE:]
    dt_pre = jnp.dot(dt_in, wdt_ref[...],
                     preferred_element_type=jnp.float32) + bdt_ref[...]
    dt = jnp.maximum(dt_pre, 0.0) + jnp.log1p(jnp.exp(-jnp.abs(dt_pre)))

    A = -jnp.exp(alogt_ref[...])
    da_s[...] = jnp.exp(dt[:, None, :] * A[None, :, :]).astype(jnp.bfloat16)
    db_s[...] = ((dt * u)[:, None, :] * bm[:, :, None]).astype(jnp.bfloat16)

    @pl.when(nc == 0)
    def _():
        h_carry[...] = jnp.zeros((D_STATE, D_INNER), jnp.bfloat16)

    def body(t, h):
        h = da_s[t] * h + db_s[t]
        hs_s[t] = h
        return h

    h_fin = jax.lax.fori_loop(0, T, body, h_carry[...], unroll=8)
    h_carry[...] = h_fin

    cm3 = jnp.broadcast_to(cm[:, :, None], (T, D_STATE, D_INNER))
    prod = (hs_s[...] * cm3.astype(jnp.bfloat16)).reshape(T * D_STATE, D_INNER)
    y_scan = jnp.dot(q_ref[...], prod, preferred_element_type=jnp.float32)
    y = (y_scan + u * d_ref[...]) * _silu(z)
    mam = jnp.dot(y, wout_ref[...], preferred_element_type=jnp.float32)
    out_ref[0] = seq + mam


def _mamba(up_l, sk_l, win_t, wconv_t, bconv, wx_t, wdt_t, bdt, alog_t, q_t,
           d_row, wout_t):
    B = up_l.shape[0]
    return pl.pallas_call(
        _mamba_kernel,
        grid=(B, NC),
        in_specs=[
            pl.BlockSpec((1, T_CHUNK, 128), lambda b, i: (b, i, 0)),
            pl.BlockSpec((1, T_CHUNK, 128), lambda b, i: (b, i, 0)),
            pl.BlockSpec((256, 1024), lambda b, i: (0, 0)),
            pl.BlockSpec((D_CONV, D_INNER), lambda b, i: (0, 0)),
            pl.BlockSpec((1, D_INNER), lambda b, i: (0, 0)),
            pl.BlockSpec((D_INNER, 48), lambda b, i: (0, 0)),
            pl.BlockSpec((DT_RANK, D_INNER), lambda b, i: (0, 0)),
            pl.BlockSpec((1, D_INNER), lambda b, i: (0, 0)),
            pl.BlockSpec((D_STATE, D_INNER), lambda b, i: (0, 0)),
            pl.BlockSpec((T_CHUNK, T_CHUNK * D_STATE), lambda b, i: (0, 0)),
            pl.BlockSpec((1, D_INNER), lambda b, i: (0, 0)),
            pl.BlockSpec((D_INNER, 256), lambda b, i: (0, 0)),
        ],
        out_specs=pl.BlockSpec((1, T_CHUNK, 256), lambda b, i: (b, i, 0)),
        out_shape=jax.ShapeDtypeStruct((B, L_FULL, 256), jnp.float32),
        scratch_shapes=[
            pltpu.VMEM((T_CHUNK + 8, D_INNER), jnp.float32),
            pltpu.VMEM((D_STATE, D_INNER), jnp.bfloat16),
            pltpu.VMEM((T_CHUNK, D_STATE, D_INNER), jnp.bfloat16),
            pltpu.VMEM((T_CHUNK, D_STATE, D_INNER), jnp.bfloat16),
            pltpu.VMEM((T_CHUNK, D_STATE, D_INNER), jnp.bfloat16),
        ],
        compiler_params=pltpu.CompilerParams(
            dimension_semantics=("parallel", "arbitrary"),
            vmem_limit_bytes=100 * 1024 * 1024),
        name="mamba_block",
    )(up_l, sk_l, win_t, wconv_t, bconv, wx_t, wdt_t, bdt, alog_t, q_t,
      d_row, wout_t)


def _conv3_kernel(x_ref, w9_ref, b_ref, me_ref, y2_ref, st_ref):
    R = 512
    bias = jnp.broadcast_to(b_ref[...], (R, 128))
    ssum = jnp.zeros((1, 128), jnp.float32)
    ssq = jnp.zeros((1, 128), jnp.float32)
    nr = L_FULL // R
    for r in range(nr):
        lo = r * R
        acc = bias
        for di in range(3):
            for dj in range(3):
                off = lo + 64 * (di - 1) + (dj - 1)
                s0, s1 = off, off + R
                c0, c1 = max(0, -s0), R - max(0, s1 - L_FULL)
                lhs = x_ref[0, s0 + c0:s1 - (R - c1), :]
                if dj == 0:
                    lhs = lhs * me_ref[0, c0:c1, :]
                elif dj == 2:
                    lhs = lhs * me_ref[1, c0:c1, :]
                part = jnp.dot(lhs, w9_ref[3 * di + dj],
                               preferred_element_type=jnp.float32)
                if c0 > 0 or c1 < R:
                    pads = []
                    if c0 > 0:
                        pads.append(jnp.zeros((c0, 128), jnp.float32))
                    pads.append(part)
                    if c1 < R:
                        pads.append(jnp.zeros((R - c1, 128), jnp.float32))
                    part = jnp.concatenate(pads, axis=0)
                acc = acc + part
        y2_ref[0, lo:lo + R, :] = acc
        ssum = ssum + jnp.sum(acc, axis=0, keepdims=True)
        ssq = ssq + jnp.sum(acc * acc, axis=0, keepdims=True)
    st_ref[0, 0:1, :] = ssum
    st_ref[0, 1:2, :] = ssq


def _conv3(xr, w9, bias, medge):
    B = xr.shape[0]
    return pl.pallas_call(
        _conv3_kernel,
        grid=(B,),
        in_specs=[
            pl.BlockSpec((1, L_FULL, 256), lambda b: (b, 0, 0)),
            pl.BlockSpec((9, 256, 128), lambda b: (0, 0, 0)),
            pl.BlockSpec((1, 128), lambda b: (0, 0)),
            pl.BlockSpec((2, 512, 256), lambda b: (0, 0, 0)),
        ],
        out_specs=[
            pl.BlockSpec((1, L_FULL, 128), lambda b: (b, 0, 0)),
            pl.BlockSpec((1, 2, 128), lambda b: (b, 0, 0)),
        ],
        out_shape=[
            jax.ShapeDtypeStruct((B, L_FULL, 128), jnp.float32),
            jax.ShapeDtypeStruct((B, 2, 128), jnp.float32),
        ],
        compiler_params=pltpu.CompilerParams(
            dimension_semantics=("parallel",),
            vmem_limit_bytes=100 * 1024 * 1024),
        name="conv3x3",
    )(xr, w9, bias, medge)


def _bn_kernel(y2_ref, st_ref, g_ref, b_ref, o_ref):
    n = 2.0 * L_FULL
    tot = st_ref[0] + st_ref[1]
    mu = tot[0:1, :] / n
    var = tot[1:2, :] / n - mu * mu
    scale = jax.lax.rsqrt(var + 1e-5) * g_ref[...]
    shift = b_ref[...] - mu * scale
    yn = y2_ref[0] * scale + shift
    o_ref[0] = yn * 0.5 * (1.0 + jax.lax.erf(yn * 0.7071067811865476))


def _bn_gelu(y2, stats, gamma, beta):
    B = y2.shape[0]
    return pl.pallas_call(
        _bn_kernel,
        grid=(B, NCONV),
        in_specs=[
            pl.BlockSpec((1, T_CONV, 128), lambda b, i: (b, i, 0)),
            pl.BlockSpec((2, 2, 128), lambda b, i: (0, 0, 0)),
            pl.BlockSpec((1, 128), lambda b, i: (0, 0)),
            pl.BlockSpec((1, 128), lambda b, i: (0, 0)),
        ],
        out_specs=pl.BlockSpec((1, T_CONV, 128), lambda b, i: (b, i, 0)),
        out_shape=jax.ShapeDtypeStruct((B, L_FULL, 128), jnp.float32),
        compiler_params=pltpu.CompilerParams(
            dimension_semantics=("parallel", "arbitrary")),
        name="bn_gelu",
    )(y2, stats, gamma, beta)


@jax.jit
def kernel(x, skip, up_w, up_b, in_proj_w, conv1d_w, conv1d_b, x_proj_w,
           dt_proj_w, dt_proj_b, A_log, D, out_proj_w, conv_w, conv_b,
           bn_gamma, bn_beta):
    B = x.shape[0]

    x_t = x.transpose(0, 2, 3, 1)
    w_stack = jnp.stack([
        up_w[:, :, 0, :].transpose(0, 2, 1).reshape(256, 256),
        up_w[:, :, 1, :].transpose(0, 2, 1).reshape(256, 256),
    ])
    bias2 = jnp.tile(up_b, 2).reshape(1, 256)
    up5 = _convt(x_t, w_stack, bias2)
    up_l = (up5.reshape(B, 2, 32, 32, 2, 128)
            .transpose(0, 2, 1, 3, 4, 5)
            .reshape(B, L_FULL, 128))
    sk_l = skip.transpose(0, 2, 3, 1).reshape(B, L_FULL, 128)

    xr = _mamba(
        up_l,
        sk_l,
        in_proj_w.T,
        conv1d_w.T,
        conv1d_b.reshape(1, D_INNER),
        x_proj_w.T,
        dt_proj_w.T,
        dt_proj_b.reshape(1, D_INNER),
        A_log.T,
        jnp.repeat(jnp.eye(T_CHUNK, dtype=jnp.bfloat16), D_STATE, axis=1),
        D.reshape(1, D_INNER),
        out_proj_w.T,
    )

    rows = jnp.arange(512, dtype=jnp.int32) % 64
    medge = jnp.stack([
        jnp.broadcast_to((rows != 0).astype(jnp.float32)[:, None], (512, 256)),
        jnp.broadcast_to((rows != 63).astype(jnp.float32)[:, None], (512, 256)),
    ])
    w9 = conv_w.transpose(2, 3, 1, 0).reshape(9, 256, 128)
    y2, stats = _conv3(xr, w9, conv_b.reshape(1, 128), medge)

    out = _bn_gelu(y2, stats, bn_gamma.reshape(1, 128),
                   bn_beta.reshape(1, 128))
    return out.reshape(B, 64, 64, 128).transpose(0, 3, 1, 2)
